# baseline (speedup 1.0000x reference)
import functools
import math

import jax
import jax.numpy as jnp
from jax import lax
from jax.experimental import pallas as pl
from jax.experimental.pallas import tpu as pltpu
from jax.experimental.pallas import tpu_sc as plsc

_LABEL_SMOOTHING = 0.1
_V = 100000
_B = 1024
_PAD = 0
_CONF = 1.0 - _LABEL_SMOOTHING
_S = _LABEL_SMOOTHING / (_V - 2)
_C_ROW = (_V - 2) * _S * math.log(_S) + _CONF * math.log(_CONF)

_SPLIT = 256

_BB = _SPLIT
_VB = 3200
_W = 4
_NVB = 8

_NW = 32
_RPW = (_B - _SPLIT) // _NW
_NG = _RPW // 8
_CHUNK = 2048
_NFULL = _V // _CHUNK
_TAIL = _V - _NFULL * _CHUNK
_TAIL_A = (_TAIL // 128) * 128
_TAIL_B = _TAIL - _TAIL_A


def _tc_body(t_ref, *refs):
    x_refs, o_ref = refs[:_W], refs[_W]
    vb = pl.program_id(0)

    @pl.when(vb == 0)
    def _init():
        o_ref[...] = jnp.zeros_like(o_ref)

    t = t_ref[...]
    nonpad = (t != _PAD).astype(jnp.float32)
    t_eff = jnp.where(t != _PAD, t, -1)
    rowpart = jnp.zeros((_BB, 1), jnp.float32)
    tsum = jnp.float32(0.0)
    for w in range(_W):
        x = x_refs[w][...]
        cols = ((vb * _W + w) * _VB
                + lax.broadcasted_iota(jnp.int32, x.shape, 1))
        xm = jnp.where(cols < _V, x, 0.0) if w == _W - 1 else x
        rowpart = rowpart + jnp.sum(xm, axis=1, keepdims=True)
        tsum = tsum + jnp.sum(jnp.where(cols == t_eff, x, 0.0))
    contrib = -_S * jnp.sum(nonpad * rowpart) - (_CONF - _S) * tsum
    corr = jnp.sum(nonpad * (_C_ROW + _S * x_refs[0][:, 0:1]))
    contrib = contrib + jnp.where(vb == 0, corr, 0.0)
    o_ref[...] = o_ref[...] + contrib


def _tc_reduce(tgt2d, output):
    def _win(w):
        return pl.BlockSpec((_BB, _VB), lambda vb, w=w: (0, vb * _W + w))

    return pl.pallas_call(
        _tc_body,
        grid=(_NVB,),
        in_specs=[pl.BlockSpec((_BB, 1), lambda vb: (0, 0))]
                 + [_win(w) for w in range(_W)],
        out_specs=pl.BlockSpec((1, 1), lambda vb: (0, 0)),
        out_shape=jax.ShapeDtypeStruct((1, 1), jnp.float32),
        compiler_params=pltpu.CompilerParams(
            dimension_semantics=("arbitrary",)),
    )(tgt2d, *([output] * _W))


def _sc_body(x_hbm, tgt_hbm, out_hbm, tgt_v, buf, tile, tails, out_v):
    wid = lax.axis_index("s") * 2 + lax.axis_index("c")
    base = _SPLIT + wid * _RPW
    pltpu.sync_copy(tgt_hbm.at[pl.ds(base, _RPW)], tgt_v.at[pl.ds(0, _RPW)])

    lane = lax.iota(jnp.int32, 16)
    is_row = lane < 8
    partial = jnp.zeros((16,), jnp.float32)

    for g in range(_NG):
        r0 = base + g * 8
        t_vec = tgt_v[pl.ds(g * 8, 16)]
        live = is_row & (t_vec != _PAD)

        def _gather_window(vals, ref, rows, start, width, t_vec=t_vec,
                           mask=live):
            off = t_vec - start
            m = mask & (off >= 0) & (off < width)
            col = jnp.where(m, off, 0)
            g16 = plsc.load_gather(ref, [rows, col], mask=m)
            return vals + jnp.where(m, g16, 0.0)

        def _chunk_step(c, carry):
            accs, gt = carry[:8], carry[8]
            pltpu.sync_copy(x_hbm.at[pl.ds(r0, 8), pl.ds(c * _CHUNK, _CHUNK)],
                            buf)
            accs = list(accs)
            for r in range(8):
                for k in range(_CHUNK // 16):
                    accs[r] = accs[r] + buf[r, pl.ds(k * 16, 16)]
            gt = _gather_window(gt, buf, lane & 7, c * _CHUNK, _CHUNK)
            return tuple(accs) + (gt,)

        zero = jnp.zeros((16,), jnp.float32)
        carry = lax.fori_loop(0, _NFULL, _chunk_step, (zero,) * 9)
        accs, gt = list(carry[:8]), carry[8]
        pltpu.sync_copy(x_hbm.at[pl.ds(r0, 8), pl.ds(_NFULL * _CHUNK, _TAIL_A)],
                        buf.at[:, pl.ds(0, _TAIL_A)])

        def _tail_step(k, accs):
            accs = list(accs)
            for r in range(8):
                accs[r] = accs[r] + buf[r, pl.ds(k * 16, 16)]
            return tuple(accs)

        accs = list(lax.fori_loop(0, _TAIL_A // 16, _tail_step, tuple(accs)))
        gt = _gather_window(gt, buf, lane & 7, _NFULL * _CHUNK, _TAIL_A)
        pltpu.sync_copy(x_hbm.at[pl.ds(r0, 8), pl.ds(_V - _TAIL_B, _TAIL_B)],
                        tails)
        for r in range(8):
            accs[r] = accs[r] + tails[r, pl.ds(0, 16)] + tails[r, pl.ds(16, 16)]
        gt = _gather_window(gt, tails, lane & 7, _V - _TAIL_B, _TAIL_B)
        pltpu.sync_copy(x_hbm.at[pl.ds(r0, 8), pl.ds(0, 128)], tile)
        g0 = plsc.load_gather(tile, [lane & 7, lane * 0], mask=is_row)
        rs = jnp.zeros((16,), jnp.float32)
        for r in range(8):
            rs = jnp.where(lane == r, jnp.sum(accs[r]), rs)
        term = (_C_ROW - _S * rs + _S * g0 - (_CONF - _S) * gt)
        partial = partial + jnp.where(live, term, 0.0)

    out_v[...] = partial
    pltpu.sync_copy(out_v, out_hbm.at[wid])


@functools.cache
def _sc_reduce():
    return pl.kernel(
        _sc_body,
        out_type=jax.ShapeDtypeStruct((_NW, 16), jnp.float32),
        mesh=plsc.VectorSubcoreMesh(core_axis_name="c", subcore_axis_name="s",
                                    num_cores=2, num_subcores=16),
        scratch_types=[
            pltpu.VMEM((_RPW + 8,), jnp.int32),
            pltpu.VMEM((8, _CHUNK), jnp.float32),
            pltpu.VMEM((8, 128), jnp.float32),
            pltpu.VMEM((8, 32), jnp.float32),
            pltpu.VMEM((16,), jnp.float32),
        ],
        compiler_params=pltpu.CompilerParams(use_tc_tiling_on_sc=True,
                                             needs_layout_passes=False),
    )


def kernel(output, target, one_hot):
    del one_hot
    tgt = target.astype(jnp.int32)
    sc_partials = _sc_reduce()(output, tgt)
    tc_loss = _tc_reduce(tgt[:_SPLIT].reshape(_SPLIT, 1), output)
    return tc_loss[0, 0] + jnp.sum(sc_partials)


# speedup vs baseline: 2.4336x; 2.4336x over previous
import math

import jax
import jax.numpy as jnp
from jax import lax
from jax.experimental import pallas as pl
from jax.experimental.pallas import tpu as pltpu

_LABEL_SMOOTHING = 0.1
_V = 100000
_B = 1024
_PAD = 0
_CONF = 1.0 - _LABEL_SMOOTHING
_S = _LABEL_SMOOTHING / (_V - 2)
_C_ROW = (_V - 2) * _S * math.log(_S) + _CONF * math.log(_CONF)

_BB = 256
_VB = 3200
_W = 4
_NVB = 8


def _tc_body(t_ref, *refs):
    x_refs, o_ref = refs[:_W], refs[_W]
    rb = pl.program_id(0)
    vb = pl.program_id(1)

    @pl.when((rb == 0) & (vb == 0))
    def _init():
        o_ref[...] = jnp.zeros_like(o_ref)

    t = t_ref[...]
    nonpad = (t != _PAD).astype(jnp.float32)
    t_eff = jnp.where(t != _PAD, t, -1)
    rowpart = jnp.zeros((_BB, 1), jnp.float32)
    tsum = jnp.float32(0.0)
    for w in range(_W):
        x = x_refs[w][...]
        cols = ((vb * _W + w) * _VB
                + lax.broadcasted_iota(jnp.int32, x.shape, 1))
        xm = jnp.where(cols < _V, x, 0.0) if w == _W - 1 else x
        rowpart = rowpart + jnp.sum(xm, axis=1, keepdims=True)
        tsum = tsum + jnp.sum(jnp.where(cols == t_eff, x, 0.0))
    contrib = -_S * jnp.sum(nonpad * rowpart) - (_CONF - _S) * tsum
    corr = jnp.sum(nonpad * (_C_ROW + _S * x_refs[0][:, 0:1]))
    contrib = contrib + jnp.where(vb == 0, corr, 0.0)
    o_ref[...] = o_ref[...] + contrib


def _tc_reduce(tgt2d, output):
    def _win(w):
        return pl.BlockSpec((_BB, _VB), lambda rb, vb, w=w: (rb, vb * _W + w))

    return pl.pallas_call(
        _tc_body,
        grid=(_B // _BB, _NVB),
        in_specs=[pl.BlockSpec((_BB, 1), lambda rb, vb: (rb, 0))]
                 + [_win(w) for w in range(_W)],
        out_specs=pl.BlockSpec((1, 1), lambda rb, vb: (0, 0)),
        out_shape=jax.ShapeDtypeStruct((1, 1), jnp.float32),
        compiler_params=pltpu.CompilerParams(
            dimension_semantics=("arbitrary", "arbitrary")),
    )(tgt2d, *([output] * _W))


def kernel(output, target, one_hot):
    del one_hot
    tgt = target.astype(jnp.int32)
    loss = _tc_reduce(tgt.reshape(_B, 1), output)
    return loss[0, 0]
